# baseline (speedup 1.0000x reference)
import functools

import jax
import jax.numpy as jnp
from jax import lax
from jax.experimental import pallas as pl
from jax.experimental.pallas import tpu as pltpu
from jax.experimental.pallas import tpu_sc as plsc

_B = 16384
_D = 64
_DP = 128
_L = 50
_LP = 52
_NG = _LP // 4
_TB = 256
_CH = 128
_NW = 32


def _sc_gather(news_pairs, user_pairs, hist_idx, news_idx, user_idx):
    bh = hist_idx.shape[0]
    n_h = bh // (_NW * _CH)
    n_b = _B // (_NW * _CH)
    mesh = plsc.VectorSubcoreMesh(core_axis_name="c", subcore_axis_name="s")
    out_types = (
        jax.ShapeDtypeStruct((bh, _DP), jnp.float32),
        jax.ShapeDtypeStruct((_B, _DP), jnp.float32),
        jax.ShapeDtypeStruct((_B, _DP), jnp.float32),
    )

    @functools.partial(
        pl.kernel,
        mesh=mesh,
        out_type=out_types,
        scratch_types=[
            pltpu.VMEM((_CH,), jnp.int32),
            pltpu.VMEM((_CH, _DP), jnp.float32),
            pltpu.SemaphoreType.DMA,
        ],
    )
    def k(news_t, user_t, hidx, nidx, uidx, out_h, out_n, out_u, idx_v, rows_v, sem):
        wid = lax.axis_index("s") * 2 + lax.axis_index("c")

        def chunk(table, idx_hbm, out_hbm, nchunks, i):
            base = (wid * nchunks + i) * _CH
            pltpu.sync_copy(idx_hbm.at[pl.ds(base, _CH)], idx_v)
            pltpu.async_copy(table.at[idx_v], rows_v, sem).wait()
            pltpu.sync_copy(rows_v, out_hbm.at[pl.ds(base, _CH)])

        @pl.loop(0, n_h)
        def _(i):
            chunk(news_t, hidx, out_h, n_h, i)

        @pl.loop(0, n_b)
        def _(i):
            chunk(news_t, nidx, out_n, n_b, i)

        @pl.loop(0, n_b)
        def _(i):
            chunk(user_t, uidx, out_u, n_b, i)

    return k(news_pairs, user_pairs, hist_idx, news_idx, user_idx)


def _half(pair, idx_col):
    odd = (idx_col & 1) == 1
    return jnp.where(odd, pair[:, _D:], pair[:, :_D])


def _tc_body(hist_ref, hidx_ref, upair_ref, uid_ref, npair_ref, nid_ref,
             w1s_ref, b1s_ref, w2_ref, b2_ref, wu_ref, bu_ref, wn_ref,
             bn_ref, out_ref):
    w2 = w2_ref[...]
    b2 = b2_ref[...]
    hidx = hidx_ref[...]
    neg = jnp.float32(-1e9)
    m = jnp.full((_TB, 1), -1e30, jnp.float32)
    s = jnp.zeros((_TB, 1), jnp.float32)
    acc = jnp.zeros((_TB, _D), jnp.float32)
    for g in range(_NG):
        xs = []
        for kk in range(4):
            l = 4 * g + kk
            pair = hist_ref[:, l * _DP:(l + 1) * _DP]
            xs.append(_half(pair, hidx[:, l:l + 1]))
        x4 = jnp.concatenate(xs, axis=1)
        h4 = jnp.tanh(
            jnp.dot(x4, w1s_ref[...], preferred_element_type=jnp.float32)
            + b1s_ref[...])
        for kk in range(4):
            l = 4 * g + kk
            if l >= _L:
                continue
            a = jnp.sum(h4[:, kk * 32:(kk + 1) * 32] * w2, axis=1,
                        keepdims=True) + b2
            a = jnp.where(hidx[:, l:l + 1] != 0, a, neg)
            m2 = jnp.maximum(m, a)
            c = jnp.exp(m - m2)
            p = jnp.exp(a - m2)
            s = s * c + p
            acc = acc * c + p * xs[kk]
            m = m2
    hist_repr = acc / s
    uemb = _half(upair_ref[...], uid_ref[...])
    nemb = _half(npair_ref[...], nid_ref[...])
    u = uemb + hist_repr
    ur = jnp.maximum(
        jnp.dot(u, wu_ref[...], preferred_element_type=jnp.float32)
        + bu_ref[...], 0.0)
    nr = jnp.maximum(
        jnp.dot(nemb, wn_ref[...], preferred_element_type=jnp.float32)
        + bn_ref[...], 0.0)
    out_ref[...] = jax.nn.sigmoid(jnp.sum(ur * nr, axis=1, keepdims=True))


def _tc_call(hist2d, history_p, gath_u, user_idx, gath_n, news_idx,
             w1s, b1s, w2r, b2r, W_user, b_user, W_news, b_news):
    grid = _B // _TB
    return pl.pallas_call(
        _tc_body,
        grid=(grid,),
        in_specs=[
            pl.BlockSpec((_TB, _LP * _DP), lambda i: (i, 0)),
            pl.BlockSpec((_TB, _LP), lambda i: (i, 0)),
            pl.BlockSpec((_TB, _DP), lambda i: (i, 0)),
            pl.BlockSpec((_TB, 1), lambda i: (i, 0)),
            pl.BlockSpec((_TB, _DP), lambda i: (i, 0)),
            pl.BlockSpec((_TB, 1), lambda i: (i, 0)),
            pl.BlockSpec((4 * _D, 128), lambda i: (0, 0)),
            pl.BlockSpec((1, 128), lambda i: (0, 0)),
            pl.BlockSpec((1, 32), lambda i: (0, 0)),
            pl.BlockSpec((1, 1), lambda i: (0, 0)),
            pl.BlockSpec((_D, _D), lambda i: (0, 0)),
            pl.BlockSpec((1, _D), lambda i: (0, 0)),
            pl.BlockSpec((_D, _D), lambda i: (0, 0)),
            pl.BlockSpec((1, _D), lambda i: (0, 0)),
        ],
        out_specs=pl.BlockSpec((_TB, 1), lambda i: (i, 0)),
        out_shape=jax.ShapeDtypeStruct((_B, 1), jnp.float32),
    )(hist2d, history_p, gath_u, user_idx.reshape(_B, 1),
      gath_n, news_idx.reshape(_B, 1), w1s, b1s, w2r, b2r,
      W_user, b_user[None, :], W_news, b_news[None, :])


def kernel(user_idx, news_idx, history, user_table, news_table, W_user,
           b_user, W_news, b_news, W_a1, b_a1, W_a2, b_a2):
    history_p = jnp.concatenate(
        [history, jnp.zeros((_B, _LP - _L), history.dtype)], axis=1)
    hist_pair_idx = (history_p >> 1).reshape(-1)

    news_pairs = news_table.reshape(news_table.shape[0] // 2, _DP)
    user_pairs = user_table.reshape(user_table.shape[0] // 2, _DP)

    gath_h, gath_n, gath_u = _sc_gather(
        news_pairs, user_pairs, hist_pair_idx, news_idx >> 1, user_idx >> 1)
    hist2d = gath_h.reshape(_B, _LP * _DP)

    w1s = jnp.kron(jnp.eye(4, dtype=jnp.float32), W_a1)
    b1s = jnp.tile(b_a1, 4)[None, :]
    w2r = W_a2[:, 0][None, :]
    b2r = b_a2.reshape(1, 1)

    out = _tc_call(hist2d, history_p, gath_u, user_idx, gath_n, news_idx,
                   w1s, b1s, w2r, b2r, W_user, b_user, W_news, b_news)
    return out[:, 0]
